# baseline (speedup 1.0000x reference)
import functools

import jax
import jax.numpy as jnp
from jax import lax
from jax.experimental import pallas as pl
from jax.experimental.pallas import tpu as pltpu
from jax.experimental.pallas import tpu_sc as plsc

NC = 2
NS = 16
NW = NC * NS
K = 16


def _gather_body(table_hbm, idx_hbm, out_hbm, idx_v, rowbuf, sems, *,
                 b_per_w, n_groups):
    wid = lax.axis_index("s") * NC + lax.axis_index("c")
    base = wid * b_per_w
    pltpu.sync_copy(idx_hbm.at[wid], idx_v)

    def fire(g, buf):
        iv = idx_v[pl.ds(g * K, K)]
        for i in range(K):
            pltpu.async_copy(table_hbm.at[iv[i]],
                             rowbuf.at[buf, i], sems.at[buf])

    def drain(buf):
        pltpu.make_async_copy(table_hbm.at[pl.ds(0, K)],
                              rowbuf.at[buf], sems.at[buf]).wait()

    def put(g, buf):
        pltpu.sync_copy(rowbuf.at[buf],
                        out_hbm.at[pl.ds(base + g * K, K)])

    fire(0, 0)

    def loop_body(h, _):
        g0 = 2 * h

        @pl.when(g0 + 1 < n_groups)
        def _():
            fire(g0 + 1, 1)

        drain(0)
        put(g0, 0)

        @pl.when(g0 + 2 < n_groups)
        def _():
            fire(g0 + 2, 0)

        @pl.when(g0 + 1 < n_groups)
        def _():
            drain(1)
            put(g0 + 1, 1)

        return ()

    lax.fori_loop(0, (n_groups + 1) // 2, loop_body, ())


def kernel(idx, data):
    (B,) = idx.shape
    V, D = data.shape
    b_per_w = B // NW
    n_groups = b_per_w // K
    idx2 = idx.astype(jnp.int32).reshape(NW, b_per_w)

    mesh = plsc.VectorSubcoreMesh(core_axis_name="c", subcore_axis_name="s")
    k = functools.partial(
        pl.kernel,
        mesh=mesh,
        out_type=jax.ShapeDtypeStruct((B, D), jnp.float32),
        scratch_types=[
            pltpu.VMEM((b_per_w,), jnp.int32),
            pltpu.VMEM((2, K, D), jnp.float32),
            pltpu.SemaphoreType.DMA((2,)),
        ],
        compiler_params=pltpu.CompilerParams(use_tc_tiling_on_sc=True),
    )(functools.partial(_gather_body, b_per_w=b_per_w, n_groups=n_groups))
    return k(data, idx2)


# speedup vs baseline: 1.4856x; 1.4856x over previous
import functools

import jax
import jax.numpy as jnp
from jax import lax
from jax.experimental import pallas as pl
from jax.experimental.pallas import tpu as pltpu
from jax.experimental.pallas import tpu_sc as plsc

NC = 2
NS = 16
NW = NC * NS
G = 4
LANES = 16


def _gather_body(tab_hbm, idx_hbm, out_hbm, idx_v, blocks, rowbuf, sems, *,
                 b_per_w, n_groups, D):
    wid = lax.axis_index("s") * NC + lax.axis_index("c")
    base = wid * b_per_w
    pltpu.sync_copy(idx_hbm.at[wid], idx_v.at[pl.ds(0, b_per_w)])

    def fire(g, buf):
        iv = idx_v[pl.ds(g * G, LANES)]
        for j in range(G):
            off = pl.multiple_of(
                lax.shift_left(lax.shift_right_logical(iv[j], 7), 7), 128)
            pltpu.async_copy(tab_hbm.at[:, pl.ds(off, 128)],
                             blocks.at[buf, j], sems.at[buf])

    def drain(buf):
        for j in range(G):
            pltpu.make_async_copy(tab_hbm.at[:, pl.ds(0, 128)],
                                  blocks.at[buf, j], sems.at[buf]).wait()

    def extract(g, buf):
        iv = idx_v[pl.ds(g * G, LANES)]
        lanes = jnp.bitwise_and(iv, 127)
        for j in range(G):
            lv = jnp.full((LANES,), lanes[j], jnp.int32)
            r = (g % 2) * G + j
            for v in range(D // LANES):
                sub = jnp.arange(LANES, dtype=jnp.int32) + (v * LANES)
                x = plsc.load_gather(
                    blocks,
                    [jnp.full((LANES,), buf, jnp.int32),
                     jnp.full((LANES,), j, jnp.int32), sub, lv])
                rowbuf[r, pl.ds(v * LANES, LANES)] = x

    def put(g):
        pltpu.sync_copy(rowbuf,
                        out_hbm.at[pl.ds(base + (g - 1) * G, 2 * G)])

    fire(0, 0)

    def loop_body(h, _):
        g0 = 2 * h

        @pl.when(g0 + 1 < n_groups)
        def _():
            fire(g0 + 1, 1)

        drain(0)
        extract(g0, 0)

        @pl.when(g0 + 2 < n_groups)
        def _():
            fire(g0 + 2, 0)

        @pl.when(g0 + 1 < n_groups)
        def _():
            drain(1)
            extract(g0 + 1, 1)
            put(g0 + 1)

        return ()

    lax.fori_loop(0, (n_groups + 1) // 2, loop_body, ())


def kernel(idx, data):
    (B,) = idx.shape
    V, D = data.shape
    b_per_w = B // NW
    n_groups = b_per_w // G
    idx2 = idx.astype(jnp.int32).reshape(NW, b_per_w)

    mesh = plsc.VectorSubcoreMesh(core_axis_name="c", subcore_axis_name="s")
    k = functools.partial(
        pl.kernel,
        mesh=mesh,
        out_type=jax.ShapeDtypeStruct((B, D), jnp.float32),
        scratch_types=[
            pltpu.VMEM((b_per_w + LANES,), jnp.int32),
            pltpu.VMEM((2, G, D, 128), jnp.float32),
            pltpu.VMEM((2 * G, D), jnp.float32),
            pltpu.SemaphoreType.DMA((2,)),
        ],
        compiler_params=pltpu.CompilerParams(needs_layout_passes=False),
    )(functools.partial(_gather_body, b_per_w=b_per_w, n_groups=n_groups,
                        D=D))
    return k(data.T, idx2)


# speedup vs baseline: 2.2336x; 1.5034x over previous
import functools

import jax
import jax.numpy as jnp
from jax import lax
from jax.experimental import pallas as pl
from jax.experimental.pallas import tpu as pltpu
from jax.experimental.pallas import tpu_sc as plsc

NC = 2
NS = 16
NW = NC * NS
SB = 256
LANES = 16
ROWSLOTS = 16


def _splat(x):
    return jnp.full((LANES,), x, jnp.int32)


def _iota():
    return jnp.arange(LANES, dtype=jnp.int32)


def _gather_body(tab_hbm, idx_hbm, out_hbm, idx_all, matched, cur, tlist,
                 blocks, tailblk, rowslots, sems, rowsem, *, B, V, D):
    tail0 = (V // 128) * 128
    n_units = tail0 // SB
    wid = lax.axis_index("s") * NC + lax.axis_index("c")
    u_lo = (wid * n_units) // NW
    u_hi = ((wid + 1) * n_units) // NW
    c0 = u_lo * SB
    n_sb = u_hi - u_lo
    span = n_sb * SB

    pltpu.sync_copy(idx_hbm, idx_all)
    tailcopy = pltpu.async_copy(tab_hbm.at[:, pl.ds(tail0, V - tail0)],
                                tailblk, rowsem)

    def scan_body(v, carry):
        mcnt, tcnt = carry
        x = idx_all[pl.ds(v * LANES, LANES)]
        pos = _iota() + (v * LANES)
        xl = x - c0
        mmask = jnp.logical_and(xl >= 0, xl < span)
        e = jnp.bitwise_or(lax.shift_left(xl, 14), pos)
        plsc.store_compressed(matched.at[pl.ds(mcnt, LANES)], e, mask=mmask)
        mc = plsc.all_reduce_population_count(mmask)[0]
        tmask = jnp.logical_and(x >= tail0,
                                jnp.bitwise_and(pos, NW - 1) == wid)
        e2 = jnp.bitwise_or(lax.shift_left(x - tail0, 14), pos)
        plsc.store_compressed(tlist.at[pl.ds(tcnt, LANES)], e2, mask=tmask)
        tc = plsc.all_reduce_population_count(tmask)[0]
        return mcnt + mc, tcnt + tc

    m, tcnt = lax.fori_loop(0, B // LANES, scan_body, (0, 0), unroll=2)

    def emit_row(e, blk_idx_fn, nout):
        off = jnp.bitwise_and(lax.shift_right_logical(e, 14), 0x7FFF)
        p = jnp.bitwise_and(e, 0x3FFF)
        slot = lax.rem(nout, ROWSLOTS)

        @pl.when(nout >= ROWSLOTS)
        def _():
            pltpu.make_async_copy(out_hbm.at[pl.ds(0, 1)],
                                  rowslots.at[pl.ds(0, 1)], rowsem).wait()

        for g in range(D // LANES):
            ref, idxs = blk_idx_fn(_iota() + g * LANES, _splat(off))
            x = plsc.load_gather(ref, idxs)
            rowslots[slot, 0, pl.ds(g * LANES, LANES)] = x
        pltpu.async_copy(rowslots.at[slot], out_hbm.at[pl.ds(p, 1)], rowsem)
        return nout + 1

    tailcopy.wait()

    def tail_body(j, nout):
        e = tlist[pl.ds(j, LANES)][0]
        return emit_row(e, lambda sub, offv: (tailblk, [sub, offv]), nout)

    nout = lax.fori_loop(0, tcnt, tail_body, 0)

    def fire(sb, slot):
        pltpu.async_copy(tab_hbm.at[:, pl.ds(c0 + sb * SB, SB)],
                         blocks.at[slot], sems.at[slot])

    @pl.when(n_sb > 0)
    def _():
        fire(0, 0)

    @pl.when(n_sb > 1)
    def _():
        fire(1, 1)

    def sb_body(sb, nout):
        slot = lax.rem(sb, 2)
        pltpu.make_async_copy(tab_hbm.at[:, pl.ds(0, SB)],
                              blocks.at[slot], sems.at[slot]).wait()

        def rescan(u, c):
            e = matched[pl.ds(u * LANES, LANES)]
            valid = (_iota() + u * LANES) < m
            sel = jnp.logical_and(
                lax.shift_right_logical(e, 14 + 8) == sb, valid)
            plsc.store_compressed(cur.at[pl.ds(c, LANES)], e, mask=sel)
            return c + plsc.all_reduce_population_count(sel)[0]

        c = lax.fori_loop(0, (m + LANES - 1) // LANES, rescan, 0)

        def ex_body(j, nout):
            e = cur[pl.ds(j, LANES)][0]
            eo = jnp.bitwise_and(e, (1 << 22) - 1)
            return emit_row(
                eo,
                lambda sub, offv: (blocks, [_splat(slot), sub, offv]),
                nout)

        nout = lax.fori_loop(0, c, ex_body, nout)

        @pl.when(sb + 2 < n_sb)
        def _():
            fire(sb + 2, slot)

        return nout

    nout = lax.fori_loop(0, n_sb, sb_body, nout)

    def drain_body(_, carry):
        pltpu.make_async_copy(out_hbm.at[pl.ds(0, 1)],
                              rowslots.at[pl.ds(0, 1)], rowsem).wait()
        return carry

    lax.fori_loop(0, jnp.minimum(nout, ROWSLOTS), drain_body, 0)


def kernel(idx, data):
    (B,) = idx.shape
    V, D = data.shape
    idx1 = idx.astype(jnp.int32)
    tail_w = V - (V // 128) * 128

    mesh = plsc.VectorSubcoreMesh(core_axis_name="c", subcore_axis_name="s")
    k = functools.partial(
        pl.kernel,
        mesh=mesh,
        out_type=jax.ShapeDtypeStruct((B, D), jnp.float32),
        scratch_types=[
            pltpu.VMEM((B,), jnp.int32),
            pltpu.VMEM((B + LANES,), jnp.int32),
            pltpu.VMEM((B + LANES,), jnp.int32),
            pltpu.VMEM((B + LANES,), jnp.int32),
            pltpu.VMEM((2, D, SB), jnp.float32),
            pltpu.VMEM((D, tail_w), jnp.float32),
            pltpu.VMEM((ROWSLOTS, 1, D), jnp.float32),
            pltpu.SemaphoreType.DMA((2,)),
            pltpu.SemaphoreType.DMA,
        ],
        compiler_params=pltpu.CompilerParams(needs_layout_passes=False),
    )(functools.partial(_gather_body, B=B, V=V, D=D))
    return k(data.T, idx1)


# speedup vs baseline: 2.5941x; 1.1614x over previous
import functools

import jax
import jax.numpy as jnp
from jax import lax
from jax.experimental import pallas as pl
from jax.experimental.pallas import tpu as pltpu
from jax.experimental.pallas import tpu_sc as plsc

NC = 2
NS = 16
NW = NC * NS
SB = 512
SB_BITS = 9
LANES = 16
ROWSLOTS = 16


def _splat(x):
    return jnp.full((LANES,), x, jnp.int32)


def _iota():
    return jnp.arange(LANES, dtype=jnp.int32)


def _gather_body(tab_hbm, idx_hbm, out_hbm, idx_all, matched, cur, tlist,
                 blocks, tailblk, rowslots, sems, rowsem, *, B, V, D):
    tail0 = (V // 128) * 128
    n_units = tail0 // SB
    wid = lax.axis_index("s") * NC + lax.axis_index("c")
    u_lo = (wid * n_units) // NW
    u_hi = ((wid + 1) * n_units) // NW
    c0 = u_lo * SB
    n_sb = u_hi - u_lo
    span = n_sb * SB

    pltpu.sync_copy(idx_hbm, idx_all)
    tailcopy = pltpu.async_copy(tab_hbm.at[:, pl.ds(tail0, V - tail0)],
                                tailblk, rowsem)

    def fire(sb, slot):
        pltpu.async_copy(tab_hbm.at[:, pl.ds(c0 + sb * SB, SB)],
                         blocks.at[slot], sems.at[slot])

    @pl.when(n_sb > 0)
    def _():
        fire(0, 0)

    @pl.when(n_sb > 1)
    def _():
        fire(1, 1)

    def scan_body(v, carry):
        mcnt, tcnt = carry
        x = idx_all[pl.ds(v * LANES, LANES)]
        pos = _iota() + (v * LANES)
        xl = x - c0
        mmask = jnp.logical_and(xl >= 0, xl < span)
        e = jnp.bitwise_or(lax.shift_left(xl, 14), pos)
        plsc.store_compressed(matched.at[pl.ds(mcnt, LANES)], e, mask=mmask)
        mc = plsc.all_reduce_population_count(mmask)[0]
        tmask = jnp.logical_and(x >= tail0,
                                jnp.bitwise_and(pos, NW - 1) == wid)
        e2 = jnp.bitwise_or(lax.shift_left(x - tail0, 14), pos)
        plsc.store_compressed(tlist.at[pl.ds(tcnt, LANES)], e2, mask=tmask)
        tc = plsc.all_reduce_population_count(tmask)[0]
        return mcnt + mc, tcnt + tc

    m, tcnt = lax.fori_loop(0, B // LANES, scan_body, (0, 0), unroll=2)

    def emit_row(e, blk_idx_fn, nout):
        off = jnp.bitwise_and(lax.shift_right_logical(e, 14), 0x7FFF)
        p = jnp.bitwise_and(e, 0x3FFF)
        slot = lax.rem(nout, ROWSLOTS)

        @pl.when(nout >= ROWSLOTS)
        def _():
            pltpu.make_async_copy(out_hbm.at[pl.ds(0, 1)],
                                  rowslots.at[pl.ds(0, 1)], rowsem).wait()

        for g in range(D // LANES):
            ref, idxs = blk_idx_fn(_iota() + g * LANES, _splat(off))
            x = plsc.load_gather(ref, idxs)
            rowslots[slot, 0, pl.ds(g * LANES, LANES)] = x
        pltpu.async_copy(rowslots.at[slot], out_hbm.at[pl.ds(p, 1)], rowsem)
        return nout + 1

    tailcopy.wait()

    def tail_body(j, nout):
        e = tlist[pl.ds(j, LANES)][0]
        return emit_row(e, lambda sub, offv: (tailblk, [sub, offv]), nout)

    nout = lax.fori_loop(0, tcnt, tail_body, 0)

    def sb_body(sb, nout):
        slot = lax.rem(sb, 2)
        pltpu.make_async_copy(tab_hbm.at[:, pl.ds(0, SB)],
                              blocks.at[slot], sems.at[slot]).wait()

        def rescan(u, c):
            e = matched[pl.ds(u * LANES, LANES)]
            valid = (_iota() + u * LANES) < m
            sel = jnp.logical_and(
                lax.shift_right_logical(e, 14 + SB_BITS) == sb, valid)
            plsc.store_compressed(cur.at[pl.ds(c, LANES)], e, mask=sel)
            return c + plsc.all_reduce_population_count(sel)[0]

        c = lax.fori_loop(0, (m + LANES - 1) // LANES, rescan, 0)

        def ex_body(j, nout):
            e = cur[pl.ds(j, LANES)][0]
            eo = jnp.bitwise_and(e, (1 << (14 + SB_BITS)) - 1)
            return emit_row(
                eo,
                lambda sub, offv: (blocks, [_splat(slot), sub, offv]),
                nout)

        nout = lax.fori_loop(0, c, ex_body, nout)

        @pl.when(sb + 2 < n_sb)
        def _():
            fire(sb + 2, slot)

        return nout

    nout = lax.fori_loop(0, n_sb, sb_body, nout)

    def drain_body(_, carry):
        pltpu.make_async_copy(out_hbm.at[pl.ds(0, 1)],
                              rowslots.at[pl.ds(0, 1)], rowsem).wait()
        return carry

    lax.fori_loop(0, jnp.minimum(nout, ROWSLOTS), drain_body, 0)


def kernel(idx, data):
    (B,) = idx.shape
    V, D = data.shape
    idx1 = idx.astype(jnp.int32)
    tail_w = V - (V // 128) * 128

    mesh = plsc.VectorSubcoreMesh(core_axis_name="c", subcore_axis_name="s")
    k = functools.partial(
        pl.kernel,
        mesh=mesh,
        out_type=jax.ShapeDtypeStruct((B, D), jnp.float32),
        scratch_types=[
            pltpu.VMEM((B,), jnp.int32),
            pltpu.VMEM((B + LANES,), jnp.int32),
            pltpu.VMEM((B + LANES,), jnp.int32),
            pltpu.VMEM((B // NW + LANES,), jnp.int32),
            pltpu.VMEM((2, D, SB), jnp.float32),
            pltpu.VMEM((D, tail_w), jnp.float32),
            pltpu.VMEM((ROWSLOTS, 1, D), jnp.float32),
            pltpu.SemaphoreType.DMA((2,)),
            pltpu.SemaphoreType.DMA,
        ],
        compiler_params=pltpu.CompilerParams(needs_layout_passes=False),
    )(functools.partial(_gather_body, B=B, V=V, D=D))
    return k(data.T, idx1)


# speedup vs baseline: 2.7460x; 1.0585x over previous
import functools

import jax
import jax.numpy as jnp
from jax import lax
from jax.experimental import pallas as pl
from jax.experimental.pallas import tpu as pltpu
from jax.experimental.pallas import tpu_sc as plsc

NC = 2
NS = 16
NW = NC * NS
SB = 512
SB_BITS = 9
LANES = 16
ROWSLOTS = 32


def _splat(x):
    return jnp.full((LANES,), x, jnp.int32)


def _iota():
    return jnp.arange(LANES, dtype=jnp.int32)


def _gather_body(tab_hbm, idx_hbm, out_hbm, idx_all, matched, cur, tlist,
                 blocks, tailblk, rowslots, sems, rowsem, *, B, V, D):
    tail0 = (V // 128) * 128
    n_units = tail0 // SB
    wid = lax.axis_index("s") * NC + lax.axis_index("c")
    u_lo = (wid * n_units) // NW
    u_hi = ((wid + 1) * n_units) // NW
    c0 = u_lo * SB
    n_sb = u_hi - u_lo
    span = n_sb * SB

    pltpu.sync_copy(idx_hbm, idx_all)
    tailcopy = pltpu.async_copy(tab_hbm.at[:, pl.ds(tail0, V - tail0)],
                                tailblk, rowsem)

    def fire(sb, slot):
        pltpu.async_copy(tab_hbm.at[:, pl.ds(c0 + sb * SB, SB)],
                         blocks.at[slot], sems.at[slot])

    @pl.when(n_sb > 0)
    def _():
        fire(0, 0)

    @pl.when(n_sb > 1)
    def _():
        fire(1, 1)

    def scan_body(v, carry):
        mcnt, tcnt = carry
        x = idx_all[pl.ds(v * LANES, LANES)]
        pos = _iota() + (v * LANES)
        xl = x - c0
        mmask = jnp.logical_and(xl >= 0, xl < span)
        e = jnp.bitwise_or(lax.shift_left(xl, 14), pos)
        plsc.store_compressed(matched.at[pl.ds(mcnt, LANES)], e, mask=mmask)
        mc = plsc.all_reduce_population_count(mmask)[0]
        tmask = jnp.logical_and(x >= tail0,
                                jnp.bitwise_and(pos, NW - 1) == wid)
        e2 = jnp.bitwise_or(lax.shift_left(x - tail0, 14), pos)
        plsc.store_compressed(tlist.at[pl.ds(tcnt, LANES)], e2, mask=tmask)
        tc = plsc.all_reduce_population_count(tmask)[0]
        return mcnt + mc, tcnt + tc

    m, tcnt = lax.fori_loop(0, B // LANES, scan_body, (0, 0), unroll=4)

    def emit_row(e, blk_idx_fn, nout):
        off = jnp.bitwise_and(lax.shift_right_logical(e, 14), 0x7FFF)
        p = jnp.bitwise_and(e, 0x3FFF)
        slot = lax.rem(nout, ROWSLOTS)

        @pl.when(nout >= ROWSLOTS)
        def _():
            pltpu.make_async_copy(out_hbm.at[pl.ds(0, 1)],
                                  rowslots.at[pl.ds(0, 1)], rowsem).wait()

        for g in range(D // LANES):
            ref, idxs = blk_idx_fn(_iota() + g * LANES, _splat(off))
            x = plsc.load_gather(ref, idxs)
            rowslots[slot, 0, pl.ds(g * LANES, LANES)] = x
        pltpu.async_copy(rowslots.at[slot], out_hbm.at[pl.ds(p, 1)], rowsem)
        return nout + 1

    tailcopy.wait()

    def tail_body(j, nout):
        e = tlist[pl.ds(j, LANES)][0]
        return emit_row(e, lambda sub, offv: (tailblk, [sub, offv]), nout)

    nout = lax.fori_loop(0, tcnt, tail_body, 0)

    def sb_body(sb, nout):
        slot = lax.rem(sb, 2)

        def rescan(u, c):
            e = matched[pl.ds(u * LANES, LANES)]
            valid = (_iota() + u * LANES) < m
            sel = jnp.logical_and(
                lax.shift_right_logical(e, 14 + SB_BITS) == sb, valid)
            plsc.store_compressed(cur.at[pl.ds(c, LANES)], e, mask=sel)
            return c + plsc.all_reduce_population_count(sel)[0]

        c = lax.fori_loop(0, (m + LANES - 1) // LANES, rescan, 0)

        pltpu.make_async_copy(tab_hbm.at[:, pl.ds(0, SB)],
                              blocks.at[slot], sems.at[slot]).wait()

        def ex_body(j, nout):
            e = cur[pl.ds(j, LANES)][0]
            eo = jnp.bitwise_and(e, (1 << (14 + SB_BITS)) - 1)
            return emit_row(
                eo,
                lambda sub, offv: (blocks, [_splat(slot), sub, offv]),
                nout)

        nout = lax.fori_loop(0, c, ex_body, nout)

        @pl.when(sb + 2 < n_sb)
        def _():
            fire(sb + 2, slot)

        return nout

    nout = lax.fori_loop(0, n_sb, sb_body, nout)

    def drain_body(_, carry):
        pltpu.make_async_copy(out_hbm.at[pl.ds(0, 1)],
                              rowslots.at[pl.ds(0, 1)], rowsem).wait()
        return carry

    lax.fori_loop(0, jnp.minimum(nout, ROWSLOTS), drain_body, 0)


def kernel(idx, data):
    (B,) = idx.shape
    V, D = data.shape
    idx1 = idx.astype(jnp.int32)
    tail_w = V - (V // 128) * 128

    mesh = plsc.VectorSubcoreMesh(core_axis_name="c", subcore_axis_name="s")
    k = functools.partial(
        pl.kernel,
        mesh=mesh,
        out_type=jax.ShapeDtypeStruct((B, D), jnp.float32),
        scratch_types=[
            pltpu.VMEM((B,), jnp.int32),
            pltpu.VMEM((B + LANES,), jnp.int32),
            pltpu.VMEM((B + LANES,), jnp.int32),
            pltpu.VMEM((B // NW + LANES,), jnp.int32),
            pltpu.VMEM((2, D, SB), jnp.float32),
            pltpu.VMEM((D, tail_w), jnp.float32),
            pltpu.VMEM((ROWSLOTS, 1, D), jnp.float32),
            pltpu.SemaphoreType.DMA((2,)),
            pltpu.SemaphoreType.DMA,
        ],
        compiler_params=pltpu.CompilerParams(needs_layout_passes=False),
    )(functools.partial(_gather_body, B=B, V=V, D=D))
    return k(data.T, idx1)
